# baseline (speedup 1.0000x reference)
import functools

import jax
import jax.numpy as jnp
from jax import lax
from jax.experimental import pallas as pl
from jax.experimental.pallas import tpu as pltpu
from jax.experimental.pallas import tpu_sc as plsc

_N = 10000
_E = 320000
_DN = 128
_DE = 16
_HID = 128
_OUT = 16
_NG = 64
_EPS = 1e-5

_NC = 2
_NS = 16
_NW = _NC * _NS
_L = 16
_C = 80
_NCH = _E // _C
_T = _NCH // _NW
_NP = 10240
_STR = _NP // _NS
_NB = _HID // _L


def _y_body(x_ref, w_ref, b_ref, o_ref):
    o_ref[...] = (
        jnp.dot(x_ref[...], w_ref[...], preferred_element_type=jnp.float32)
        + b_ref[...]
    )


_y_call = pl.pallas_call(
    _y_body,
    out_shape=jax.ShapeDtypeStruct((_N, _HID), jnp.float32),
)

_ER = _E // 8
_EB = 2000


def _ep_body(ea_ref, w_ref, o_ref):
    o_ref[...] = jnp.dot(
        ea_ref[...], w_ref[...], preferred_element_type=jnp.float32
    )


_ep_call = pl.pallas_call(
    _ep_body,
    grid=(_ER // _EB,),
    in_specs=[
        pl.BlockSpec((_EB, 8 * _DE), lambda i: (i, 0)),
        pl.BlockSpec((8 * _DE, 8 * _HID), lambda i: (0, 0)),
    ],
    out_specs=pl.BlockSpec((_EB, 8 * _HID), lambda i: (i, 0)),
    out_shape=jax.ShapeDtypeStruct((_ER, 8 * _HID), jnp.float32),
)


def _edge_body(y_h, row_h, col_h, ep_h, outh_h, outc_h, outs2_h,
               rbuf0, rbuf1, cbuf0, cbuf1, cbuf2, cbuf3,
               ybuf0, ybuf1, epbuf0, epbuf1, ones, s2buf, acc_h, acc_c,
               semi0, semi1, semg0, semg1, sems0, sems1):
    cid = lax.axis_index("c")
    sid = lax.axis_index("s")
    wid = sid * _NC + cid

    rbuf = (rbuf0, rbuf1)
    cbuf = (cbuf0, cbuf1, cbuf2, cbuf3)
    ybuf = (ybuf0, ybuf1)
    epbuf = (epbuf0, epbuf1)
    semi = (semi0, semi1)
    semg = (semg0, semg1)
    sems = (sems0, sems1)

    zvec = jnp.zeros((_L,), jnp.float32)
    onevec = jnp.ones((_L,), jnp.float32)
    zivec = jnp.zeros((_L,), jnp.int32)

    def _zrow(i, carry):
        for b in range(_NB):
            ybuf1[i, pl.ds(b * _L, _L)] = zvec
        return carry

    lax.fori_loop(0, _C, _zrow, 0)

    def _z1(i, carry):
        ones[pl.ds(i * _L, _L)] = onevec
        cbuf[3][pl.ds(i * _L, _L)] = zivec
        return carry

    lax.fori_loop(0, _C // _L, _z1, 0)

    for k in range(_STR // _C):
        pltpu.sync_copy(ybuf1, acc_h.at[pl.ds(sid * _STR + k * _C, _C)])
    for k in range(_STR // _C):
        pltpu.sync_copy(ybuf1.at[0, pl.ds(0, _C)],
                        acc_c.at[pl.ds(sid * _STR + k * _C, _C)])
    plsc.subcore_barrier()

    def _base(t):
        return (wid + _NW * jnp.minimum(t, _T - 1)) * _C

    def _start_idx(t, s, c):
        b = _base(t)
        pltpu.async_copy(row_h.at[pl.ds(b, _C)], rbuf[s], semi[s])
        pltpu.async_copy(col_h.at[pl.ds(b, _C)], cbuf[c], semi[s])
        pltpu.async_copy(ep_h.at[pl.ds(b, _C)], epbuf[s], semi[s])

    def _wait_idx(s, c):
        pltpu.make_async_copy(row_h.at[pl.ds(0, _C)], rbuf[s], semi[s]).wait()
        pltpu.make_async_copy(col_h.at[pl.ds(0, _C)], cbuf[c], semi[s]).wait()
        pltpu.make_async_copy(ep_h.at[pl.ds(0, _C)], epbuf[s], semi[s]).wait()

    def _start_gather(s):
        pltpu.async_copy(y_h.at[rbuf[s]], ybuf[s], semg[s])

    def _wait_gather(s):
        pltpu.make_async_copy(ep_h.at[pl.ds(0, _C)], ybuf[s], semg[s]).wait()

    def _start_scatter(src, csrc, c, s):
        pltpu.async_copy(src, acc_h.at[cbuf[c]], sems[s], add=True)
        pltpu.async_copy(csrc, acc_c.at[cbuf[c]], sems[s], add=True)

    def _wait_scatter(s):
        pltpu.make_async_copy(ep_h.at[pl.ds(0, _C)], ybuf[s], sems[s]).wait()
        pltpu.make_async_copy(ep_h.at[0, pl.ds(0, _C)], ones, sems[s]).wait()

    def _step(t, p, c4, s2c):
        n = 1 - p
        _wait_idx(n, (c4 + 1) % 4)
        _wait_scatter(n)
        _start_gather(n)
        _wait_gather(p)

        def _row(r, s2r):
            out = []
            for b in range(_NB):
                sl = pl.ds(b * _L, _L)
                v = ybuf[p][r, sl] + epbuf[p][r, sl]
                v = jnp.maximum(v, 0.0)
                ybuf[p][r, sl] = v
                out.append(s2r[b] + v * v)
            return tuple(out)

        s2c = lax.fori_loop(0, _C, _row, s2c)

        _start_scatter(ybuf[p], ones, c4, p)
        _start_idx(t + 2, p, (c4 + 2) % 4)
        return s2c

    _start_idx(0, 0, 0)
    _start_scatter(ybuf1, ybuf1.at[0, pl.ds(0, _C)], 3, 1)
    _wait_idx(0, 0)
    _start_gather(0)
    _start_idx(1, 1, 1)

    def _quad(i, s2c):
        t0 = 4 * i
        for k in range(4):
            s2c = _step(t0 + k, k % 2, k, s2c)
        return s2c

    s2 = lax.fori_loop(0, _T // 4, _quad, tuple(zvec for _ in range(_NB)))
    s2 = _step(_T - 1, 0, 0, s2)
    _wait_idx(0, 2)
    _wait_gather(1)
    _wait_scatter(0)

    for b in range(_NB):
        s2buf[pl.ds(b * _L, _L)] = s2[b]

    plsc.subcore_barrier()

    for k in range(_STR // _C):
        sl = pl.ds(sid * _STR + k * _C, _C)
        pltpu.sync_copy(acc_h.at[sl], outh_h.at[cid, sl])
    slc = pl.ds(sid * _STR, _STR)
    pltpu.sync_copy(acc_c.at[slc], outc_h.at[cid, slc])
    pltpu.sync_copy(s2buf, outs2_h.at[wid])


_edge_call = pl.kernel(
    _edge_body,
    out_type=(
        jax.ShapeDtypeStruct((_NC, _NP, _HID), jnp.float32),
        jax.ShapeDtypeStruct((_NC, _NP), jnp.float32),
        jax.ShapeDtypeStruct((_NW, _HID), jnp.float32),
    ),
    mesh=plsc.VectorSubcoreMesh(
        core_axis_name="c", subcore_axis_name="s",
        num_cores=_NC, num_subcores=_NS,
    ),
    scratch_types=[
        pltpu.VMEM((_C,), jnp.int32),
        pltpu.VMEM((_C,), jnp.int32),
        pltpu.VMEM((_C,), jnp.int32),
        pltpu.VMEM((_C,), jnp.int32),
        pltpu.VMEM((_C,), jnp.int32),
        pltpu.VMEM((_C,), jnp.int32),
        pltpu.VMEM((_C, _HID), jnp.float32),
        pltpu.VMEM((_C, _HID), jnp.float32),
        pltpu.VMEM((_C, _HID), jnp.float32),
        pltpu.VMEM((_C, _HID), jnp.float32),
        pltpu.VMEM((_C,), jnp.float32),
        pltpu.VMEM((_HID,), jnp.float32),
        pltpu.VMEM_SHARED((_NP, _HID), jnp.float32),
        pltpu.VMEM_SHARED((_NP,), jnp.float32),
        pltpu.SemaphoreType.DMA,
        pltpu.SemaphoreType.DMA,
        pltpu.SemaphoreType.DMA,
        pltpu.SemaphoreType.DMA,
        pltpu.SemaphoreType.DMA,
        pltpu.SemaphoreType.DMA,
    ],
)


def _fin_body(acch, cnt_ref, s2p, x_ref, u_ref, b_ref, g1r, be1r, w1br, b1br,
              w2ar, b2ar, g2r, be2r, w2br, b2br, o_ref):
    seg = acch[0, : _N, :] + acch[1, : _N, :]
    cnt = cnt_ref[...]
    s_sum = jnp.sum(seg, axis=0, keepdims=True)
    s2_sum = jnp.sum(s2p[...], axis=0, keepdims=True)
    mu = s_sum / _E
    var = s2_sum / _E - mu * mu
    w = g1r[...] * lax.rsqrt(var + _EPS)
    beff = b1br[...] + jnp.dot(
        be1r[...] - mu * w, w1br[...], preferred_element_type=jnp.float32
    )
    summed = (
        jnp.dot(seg * w, w1br[...], preferred_element_type=jnp.float32)
        + cnt * beff
    )
    aggr = summed / jnp.maximum(cnt, 1.0)
    oh = (b_ref[...] == lax.broadcasted_iota(jnp.int32, (_N, _NG), 1))
    ub = jnp.dot(
        oh.astype(jnp.float32), u_ref[...], preferred_element_type=jnp.float32
    )
    h = (
        jnp.dot(x_ref[...], w2ar[: _DN, :], preferred_element_type=jnp.float32)
        + jnp.dot(aggr, w2ar[_DN : _DN + _OUT, :],
                  preferred_element_type=jnp.float32)
        + jnp.dot(ub, w2ar[_DN + _OUT :, :], preferred_element_type=jnp.float32)
        + b2ar[...]
    )
    h = jnp.maximum(h, 0.0)
    mu2 = jnp.mean(h, axis=0, keepdims=True)
    var2 = jnp.mean(h * h, axis=0, keepdims=True) - mu2 * mu2
    hn = (h - mu2) * lax.rsqrt(var2 + _EPS) * g2r[...] + be2r[...]
    o_ref[...] = (
        jnp.dot(hn, w2br[...], preferred_element_type=jnp.float32) + b2br[...]
    )


_fin_call = pl.pallas_call(
    _fin_body,
    out_shape=jax.ShapeDtypeStruct((_N, _OUT), jnp.float32),
)


def kernel(x, edge_index, edge_attr, u, batch, W1a, b1a, g1, be1, W1b, b1b,
           W2a, b2a, g2, be2, W2b, b2b):
    y = _y_call(x, W1a[: _DN], b1a[None])
    w1e = W1a[_DN:]
    wbig = (jnp.eye(8, dtype=jnp.float32)[:, None, :, None]
            * w1e[None, :, None, :]).reshape(8 * _DE, 8 * _HID)
    ep = _ep_call(edge_attr.reshape(_ER, 8 * _DE), wbig)
    ep = ep.reshape(_E, _HID)
    acc_h, acc_c, s2p = _edge_call(y, edge_index[0], edge_index[1], ep)
    cnt_col = (acc_c[0, : _N] + acc_c[1, : _N])[:, None]
    return _fin_call(
        acc_h, cnt_col, s2p, x, u, batch[:, None], g1[None], be1[None], W1b,
        b1b[None], W2a, b2a[None], g2[None], be2[None], W2b, b2b[None]
    )


# speedup vs baseline: 1.3443x; 1.3443x over previous
import functools

import jax
import jax.numpy as jnp
from jax import lax
from jax.experimental import pallas as pl
from jax.experimental.pallas import tpu as pltpu
from jax.experimental.pallas import tpu_sc as plsc

_N = 10000
_E = 320000
_DN = 128
_DE = 16
_HID = 128
_OUT = 16
_NG = 64
_EPS = 1e-5

_NC = 2
_NS = 16
_NW = _NC * _NS
_L = 16
_C = 80
_NCH = _E // _C
_T = _NCH // _NW
_NP = 10240
_STR = _NP // _NS
_NB = _HID // _L


def _y_body(x_ref, w_ref, b_ref, o_ref):
    o_ref[...] = (
        jnp.dot(x_ref[...], w_ref[...], preferred_element_type=jnp.float32)
        + b_ref[...]
    )


_y_call = pl.pallas_call(
    _y_body,
    out_shape=jax.ShapeDtypeStruct((_N, _HID), jnp.float32),
)

_ER = _E // 8
_EB = 2000


def _ep_body(ea_ref, w_ref, o_ref):
    p = jnp.dot(ea_ref[...], w_ref[...], preferred_element_type=jnp.float32)
    o_ref[...] = p.reshape(8 * _EB, _HID)


_ep_call = pl.pallas_call(
    _ep_body,
    grid=(_ER // _EB,),
    in_specs=[
        pl.BlockSpec((_EB, 8 * _DE), lambda i: (i, 0)),
        pl.BlockSpec((8 * _DE, 8 * _HID), lambda i: (0, 0)),
    ],
    out_specs=pl.BlockSpec((8 * _EB, _HID), lambda i: (i, 0)),
    out_shape=jax.ShapeDtypeStruct((_E, _HID), jnp.float32),
)


def _edge_body(y_h, row_h, col_h, ep_h, outh_h, outc_h, outs2_h,
               rbuf0, rbuf1, cbuf0, cbuf1, cbuf2, cbuf3,
               ybuf0, ybuf1, epbuf0, epbuf1, ones, s2buf, acc_h, acc_c,
               semi0, semi1, semg0, semg1, sems0, sems1):
    cid = lax.axis_index("c")
    sid = lax.axis_index("s")
    wid = sid * _NC + cid

    rbuf = (rbuf0, rbuf1)
    cbuf = (cbuf0, cbuf1, cbuf2, cbuf3)
    ybuf = (ybuf0, ybuf1)
    epbuf = (epbuf0, epbuf1)
    semi = (semi0, semi1)
    semg = (semg0, semg1)
    sems = (sems0, sems1)

    zvec = jnp.zeros((_L,), jnp.float32)
    onevec = jnp.ones((_L,), jnp.float32)
    zivec = jnp.zeros((_L,), jnp.int32)

    def _zrow(i, carry):
        for b in range(_NB):
            ybuf1[i, pl.ds(b * _L, _L)] = zvec
        return carry

    lax.fori_loop(0, _C, _zrow, 0)

    def _z1(i, carry):
        ones[pl.ds(i * _L, _L)] = onevec
        cbuf[3][pl.ds(i * _L, _L)] = zivec
        return carry

    lax.fori_loop(0, _C // _L, _z1, 0)

    for k in range(_STR // _C):
        pltpu.sync_copy(ybuf1, acc_h.at[pl.ds(sid * _STR + k * _C, _C)])
    for k in range(_STR // _C):
        pltpu.sync_copy(ybuf1.at[0, pl.ds(0, _C)],
                        acc_c.at[pl.ds(sid * _STR + k * _C, _C)])
    plsc.subcore_barrier()

    def _base(t):
        return (wid + _NW * jnp.minimum(t, _T - 1)) * _C

    def _start_idx(t, s, c):
        b = _base(t)
        pltpu.async_copy(row_h.at[pl.ds(b, _C)], rbuf[s], semi[s])
        pltpu.async_copy(col_h.at[pl.ds(b, _C)], cbuf[c], semi[s])
        pltpu.async_copy(ep_h.at[pl.ds(b, _C)], epbuf[s], semi[s])

    def _wait_idx(s, c):
        pltpu.make_async_copy(row_h.at[pl.ds(0, _C)], rbuf[s], semi[s]).wait()
        pltpu.make_async_copy(col_h.at[pl.ds(0, _C)], cbuf[c], semi[s]).wait()
        pltpu.make_async_copy(ep_h.at[pl.ds(0, _C)], epbuf[s], semi[s]).wait()

    def _start_gather(s):
        pltpu.async_copy(y_h.at[rbuf[s]], ybuf[s], semg[s])

    def _wait_gather(s):
        pltpu.make_async_copy(ep_h.at[pl.ds(0, _C)], ybuf[s], semg[s]).wait()

    def _start_scatter(src, csrc, c, s):
        pltpu.async_copy(src, acc_h.at[cbuf[c]], sems[s], add=True)
        pltpu.async_copy(csrc, acc_c.at[cbuf[c]], sems[s], add=True)

    def _wait_scatter(s):
        pltpu.make_async_copy(ep_h.at[pl.ds(0, _C)], ybuf[s], sems[s]).wait()
        pltpu.make_async_copy(ep_h.at[0, pl.ds(0, _C)], ones, sems[s]).wait()

    def _step(t, p, c4, s2c):
        n = 1 - p
        _wait_idx(n, (c4 + 1) % 4)
        _wait_scatter(n)
        _start_gather(n)
        _wait_gather(p)

        def _row(r, s2r):
            out = []
            for b in range(_NB):
                sl = pl.ds(b * _L, _L)
                v = ybuf[p][r, sl] + epbuf[p][r, sl]
                v = jnp.maximum(v, 0.0)
                ybuf[p][r, sl] = v
                out.append(s2r[b] + v * v)
            return tuple(out)

        s2c = lax.fori_loop(0, _C, _row, s2c)

        _start_scatter(ybuf[p], ones, c4, p)
        _start_idx(t + 2, p, (c4 + 2) % 4)
        return s2c

    _start_idx(0, 0, 0)
    _start_scatter(ybuf1, ybuf1.at[0, pl.ds(0, _C)], 3, 1)
    _wait_idx(0, 0)
    _start_gather(0)
    _start_idx(1, 1, 1)

    def _quad(i, s2c):
        t0 = 4 * i
        for k in range(4):
            s2c = _step(t0 + k, k % 2, k, s2c)
        return s2c

    s2 = lax.fori_loop(0, _T // 4, _quad, tuple(zvec for _ in range(_NB)))
    s2 = _step(_T - 1, 0, 0, s2)
    _wait_idx(0, 2)
    _wait_gather(1)
    _wait_scatter(0)

    for b in range(_NB):
        s2buf[pl.ds(b * _L, _L)] = s2[b]

    plsc.subcore_barrier()

    for k in range(_STR // _C):
        sl = pl.ds(sid * _STR + k * _C, _C)
        pltpu.sync_copy(acc_h.at[sl], outh_h.at[cid, sl])
    slc = pl.ds(sid * _STR, _STR)
    pltpu.sync_copy(acc_c.at[slc], outc_h.at[cid, slc])
    pltpu.sync_copy(s2buf, outs2_h.at[wid])


_edge_call = pl.kernel(
    _edge_body,
    out_type=(
        jax.ShapeDtypeStruct((_NC, _NP, _HID), jnp.float32),
        jax.ShapeDtypeStruct((_NC, _NP), jnp.float32),
        jax.ShapeDtypeStruct((_NW, _HID), jnp.float32),
    ),
    mesh=plsc.VectorSubcoreMesh(
        core_axis_name="c", subcore_axis_name="s",
        num_cores=_NC, num_subcores=_NS,
    ),
    scratch_types=[
        pltpu.VMEM((_C,), jnp.int32),
        pltpu.VMEM((_C,), jnp.int32),
        pltpu.VMEM((_C,), jnp.int32),
        pltpu.VMEM((_C,), jnp.int32),
        pltpu.VMEM((_C,), jnp.int32),
        pltpu.VMEM((_C,), jnp.int32),
        pltpu.VMEM((_C, _HID), jnp.float32),
        pltpu.VMEM((_C, _HID), jnp.float32),
        pltpu.VMEM((_C, _HID), jnp.float32),
        pltpu.VMEM((_C, _HID), jnp.float32),
        pltpu.VMEM((_C,), jnp.float32),
        pltpu.VMEM((_HID,), jnp.float32),
        pltpu.VMEM_SHARED((_NP, _HID), jnp.float32),
        pltpu.VMEM_SHARED((_NP,), jnp.float32),
        pltpu.SemaphoreType.DMA,
        pltpu.SemaphoreType.DMA,
        pltpu.SemaphoreType.DMA,
        pltpu.SemaphoreType.DMA,
        pltpu.SemaphoreType.DMA,
        pltpu.SemaphoreType.DMA,
    ],
)


def _fin_body(acch, cnt_ref, s2p, x_ref, u_ref, b_ref, g1r, be1r, w1br, b1br,
              w2ar, b2ar, g2r, be2r, w2br, b2br, o_ref):
    seg = acch[0, : _N, :] + acch[1, : _N, :]
    cnt = cnt_ref[...]
    s_sum = jnp.sum(seg, axis=0, keepdims=True)
    s2_sum = jnp.sum(s2p[...], axis=0, keepdims=True)
    mu = s_sum / _E
    var = s2_sum / _E - mu * mu
    w = g1r[...] * lax.rsqrt(var + _EPS)
    beff = b1br[...] + jnp.dot(
        be1r[...] - mu * w, w1br[...], preferred_element_type=jnp.float32
    )
    summed = (
        jnp.dot(seg * w, w1br[...], preferred_element_type=jnp.float32)
        + cnt * beff
    )
    aggr = summed / jnp.maximum(cnt, 1.0)
    oh = (b_ref[...] == lax.broadcasted_iota(jnp.int32, (_N, _NG), 1))
    ub = jnp.dot(
        oh.astype(jnp.float32), u_ref[...], preferred_element_type=jnp.float32
    )
    h = (
        jnp.dot(x_ref[...], w2ar[: _DN, :], preferred_element_type=jnp.float32)
        + jnp.dot(aggr, w2ar[_DN : _DN + _OUT, :],
                  preferred_element_type=jnp.float32)
        + jnp.dot(ub, w2ar[_DN + _OUT :, :], preferred_element_type=jnp.float32)
        + b2ar[...]
    )
    h = jnp.maximum(h, 0.0)
    mu2 = jnp.mean(h, axis=0, keepdims=True)
    var2 = jnp.mean(h * h, axis=0, keepdims=True) - mu2 * mu2
    hn = (h - mu2) * lax.rsqrt(var2 + _EPS) * g2r[...] + be2r[...]
    o_ref[...] = (
        jnp.dot(hn, w2br[...], preferred_element_type=jnp.float32) + b2br[...]
    )


_fin_call = pl.pallas_call(
    _fin_body,
    out_shape=jax.ShapeDtypeStruct((_N, _OUT), jnp.float32),
)


def kernel(x, edge_index, edge_attr, u, batch, W1a, b1a, g1, be1, W1b, b1b,
           W2a, b2a, g2, be2, W2b, b2b):
    y = _y_call(x, W1a[: _DN], b1a[None])
    w1e = W1a[_DN:]
    wbig = (jnp.eye(8, dtype=jnp.float32)[:, None, :, None]
            * w1e[None, :, None, :]).reshape(8 * _DE, 8 * _HID)
    ep = _ep_call(edge_attr.reshape(_ER, 8 * _DE), wbig)
    acc_h, acc_c, s2p = _edge_call(y, edge_index[0], edge_index[1], ep)
    cnt_col = (acc_c[0, : _N] + acc_c[1, : _N])[:, None]
    return _fin_call(
        acc_h, cnt_col, s2p, x, u, batch[:, None], g1[None], be1[None], W1b,
        b1b[None], W2a, b2a[None], g2[None], be2[None], W2b, b2b[None]
    )
